# baseline (speedup 1.0000x reference)
import jax
import jax.numpy as jnp
from jax.experimental import pallas as pl
from jax.experimental.pallas import tpu as pltpu


_BLOCK = 2048


def _moe_gate_kernel(hs_ref, gw_ref, zero_hbm, logits_ref, zbuf, sem):
    i = pl.program_id(0)
    n = pl.num_programs(0)

    @pl.when(i == 0)
    def _():
        zbuf[...] = jnp.zeros_like(zbuf)

    block = zbuf.shape[0]
    pltpu.make_async_copy(
        zbuf, zero_hbm.at[pl.ds(i * block, block), :], sem
    ).start()

    logits_ref[...] = jax.lax.dot_general(
        gw_ref[...], hs_ref[...],
        dimension_numbers=(((1,), (1,)), ((), ())),
        preferred_element_type=jnp.float32,
    )

    @pl.when(i >= 2)
    def _():
        pltpu.make_async_copy(
            zbuf, zero_hbm.at[pl.ds((i - 2) * block, block), :], sem
        ).wait()

    @pl.when(i == n - 1)
    def _():
        pltpu.make_async_copy(
            zbuf, zero_hbm.at[pl.ds((n - 2) * block, block), :], sem
        ).wait()
        pltpu.make_async_copy(
            zbuf, zero_hbm.at[pl.ds((n - 1) * block, block), :], sem
        ).wait()


def kernel(hidden_states, gate_weight):
    batch, seq, hidden = hidden_states.shape
    rows = batch * seq
    hs = hidden_states.reshape(rows, hidden)
    num_experts = gate_weight.shape[0]

    zeros, logits_t = pl.pallas_call(
        _moe_gate_kernel,
        grid=(rows // _BLOCK,),
        in_specs=[
            pl.BlockSpec((_BLOCK, hidden), lambda i: (i, 0)),
            pl.BlockSpec((num_experts, hidden), lambda i: (0, 0)),
        ],
        out_specs=[
            pl.BlockSpec(memory_space=pl.ANY),
            pl.BlockSpec((num_experts, _BLOCK), lambda i: (0, i)),
        ],
        out_shape=[
            jax.ShapeDtypeStruct((rows, hidden), hidden_states.dtype),
            jax.ShapeDtypeStruct((num_experts, rows), jnp.float32),
        ],
        scratch_shapes=[
            pltpu.VMEM((_BLOCK, hidden), jnp.float32),
            pltpu.SemaphoreType.DMA,
        ],
    )(hs, gate_weight)

    return zeros.reshape(batch, seq, hidden), logits_t.T


# speedup vs baseline: 1.0010x; 1.0010x over previous
import jax
import jax.numpy as jnp
from jax.experimental import pallas as pl
from jax.experimental.pallas import tpu as pltpu


_BLOCK = 2048


def _moe_gate_kernel(hs_ref, gw_ref, zero_hbm, logits_ref, zbuf, sem):
    i = pl.program_id(0)
    n = pl.num_programs(0)

    @pl.when(i == 0)
    def _():
        zbuf[...] = jnp.zeros_like(zbuf)

    block = zbuf.shape[0]
    pltpu.make_async_copy(
        zbuf, zero_hbm.at[pl.ds(i * block, block), :], sem
    ).start()

    logits_ref[...] = jax.lax.dot_general(
        gw_ref[...], hs_ref[...],
        dimension_numbers=(((1,), (1,)), ((), ())),
        preferred_element_type=jnp.float32,
    )

    @pl.when(i > 0)
    def _():
        pltpu.make_async_copy(
            zbuf, zero_hbm.at[pl.ds((i - 1) * block, block), :], sem
        ).wait()

    @pl.when(i == n - 1)
    def _():
        pltpu.make_async_copy(
            zbuf, zero_hbm.at[pl.ds(i * block, block), :], sem
        ).wait()


def kernel(hidden_states, gate_weight):
    batch, seq, hidden = hidden_states.shape
    rows = batch * seq
    hs = hidden_states.reshape(rows, hidden)
    num_experts = gate_weight.shape[0]

    zeros, logits_t = pl.pallas_call(
        _moe_gate_kernel,
        grid=(rows // _BLOCK,),
        in_specs=[
            pl.BlockSpec((_BLOCK, hidden), lambda i: (i, 0)),
            pl.BlockSpec((num_experts, hidden), lambda i: (0, 0)),
        ],
        out_specs=[
            pl.BlockSpec(memory_space=pl.ANY),
            pl.BlockSpec((num_experts, _BLOCK), lambda i: (0, i)),
        ],
        out_shape=[
            jax.ShapeDtypeStruct((rows, hidden), hidden_states.dtype),
            jax.ShapeDtypeStruct((num_experts, rows), jnp.float32),
        ],
        scratch_shapes=[
            pltpu.VMEM((_BLOCK, hidden), jnp.float32),
            pltpu.SemaphoreType.DMA,
        ],
    )(hs, gate_weight)

    return zeros.reshape(batch, seq, hidden), logits_t.T
